# baseline (speedup 1.0000x reference)
import jax
import jax.numpy as jnp
from jax import lax
from jax.experimental import pallas as pl
from jax.experimental.pallas import tpu as pltpu
from jax.experimental.pallas import tpu_sc as plsc

_DIM = 32
_NC, _NS = 2, 16
_NW = _NC * _NS
_C = 512
_L = 200
_B = 4096


def _emb_body(idx_hbm, table_hbm, out_hbm, idx_v, rows_v, out_t,
              semi0, semi1, semg0, semg1, semo0, semo1):
    n_rows = idx_hbm.shape[0]
    r_per_w = n_rows // _NW
    n_chunks = r_per_w // _C
    wid = lax.axis_index("s") * _NC + lax.axis_index("c")
    base = wid * r_per_w

    semi = (semi0, semi1)
    semg = (semg0, semg1)
    semo = (semo0, semo1)

    iota16 = lax.iota(jnp.int32, 16)
    qdiv8 = lax.shift_right_logical(iota16, 3)
    qmod8 = lax.bitwise_and(iota16, 7)
    iv = (qdiv8, qdiv8 + 2)
    sv = qmod8

    def idx_copy(c, s):
        return pltpu.make_async_copy(
            idx_hbm.at[pl.ds(base + c * _C, _C)], idx_v.at[s], semi[s])

    def fire_gather(s):
        pltpu.async_copy(table_hbm.at[idx_v.at[s]], rows_v.at[s], semg[s])

    def drain_gather(s):
        pltpu.make_async_copy(table_hbm.at[pl.ds(0, _C)],
                              rows_v.at[s], semg[s]).wait()

    def out_copy(c, s):
        flat0 = base + c * _C
        l = flat0 // _B
        j0 = (flat0 % _B) // 128
        return pltpu.make_async_copy(
            out_t.at[s], out_hbm.at[l, :, pl.ds(j0, 4), :, :], semo[s])

    def transpose(s):
        @pl.loop(0, _C, unroll=4)
        def _row(r):
            jjv = jnp.broadcast_to(lax.shift_right_logical(r, 7), (16,))
            rlv = jnp.broadcast_to(lax.bitwise_and(r, 127), (16,))
            for h in range(2):
                vec = rows_v[s, r, pl.ds(h * 16, 16)]
                plsc.store_scatter(out_t.at[s], [iv[h], jjv, sv, rlv], vec)

    idx_copy(0, 0).start()
    idx_copy(1, 1).start()
    idx_copy(0, 0).wait()
    fire_gather(0)

    @pl.loop(0, n_chunks // 2)
    def _pair(t):
        for b in range(2):
            c = 2 * t + b
            s = b
            drain_gather(s)

            @pl.when(c + 2 < n_chunks)
            def _():
                idx_copy(c + 2, s).start()

            @pl.when(c + 1 < n_chunks)
            def _():
                idx_copy(c + 1, 1 - s).wait()
                fire_gather(1 - s)

            @pl.when(c >= 2)
            def _():
                out_copy(c - 2, s).wait()

            transpose(s)
            out_copy(c, s).start()

    out_copy(n_chunks - 2, 0).wait()
    out_copy(n_chunks - 1, 1).wait()


def kernel(inp, table):
    b, l = inp.shape
    n = b * l
    idx = inp.T.reshape(n).astype(jnp.int32)
    mesh = plsc.VectorSubcoreMesh(core_axis_name="c", subcore_axis_name="s")
    out5d = pl.kernel(
        _emb_body,
        out_type=jax.ShapeDtypeStruct((_L, 4, 32, 8, 128), table.dtype),
        mesh=mesh,
        scratch_types=[
            pltpu.VMEM((2, _C), jnp.int32),
            pltpu.VMEM((2, _C, _DIM), jnp.float32),
            pltpu.VMEM((2, 4, 4, 8, 128), jnp.float32),
            pltpu.SemaphoreType.DMA,
            pltpu.SemaphoreType.DMA,
            pltpu.SemaphoreType.DMA,
            pltpu.SemaphoreType.DMA,
            pltpu.SemaphoreType.DMA,
            pltpu.SemaphoreType.DMA,
        ],
        compiler_params=pltpu.CompilerParams(use_tc_tiling_on_sc=False,
                                             needs_layout_passes=False),
    )(idx, table)
    return out5d.transpose(2, 4, 0, 1, 3).reshape(b, l, _DIM)
